# baseline (speedup 1.0000x reference)
import functools

import jax
import jax.numpy as jnp
from jax import lax
from jax.experimental import pallas as pl
from jax.experimental.pallas import tpu as pltpu
from jax.experimental.pallas import tpu_sc as plsc

_NC = 2
_NS = 16
_NW = _NC * _NS
_LANES = 16
_BLK = 128
_HB = 64


def _sc_aggregate(x, col_p, row_p, val_p, zeros, b_w, n_pad):
    _, d = x.shape
    rows_per_sub = n_pad // _NS
    mesh = plsc.VectorSubcoreMesh(core_axis_name="c", subcore_axis_name="s")

    @functools.partial(
        pl.kernel,
        out_type=jax.ShapeDtypeStruct((_NC, n_pad, d), jnp.float32),
        mesh=mesh,
        scratch_types=[
            pltpu.VMEM((b_w, _BLK), jnp.int32),
            pltpu.VMEM((b_w, _BLK), jnp.int32),
            pltpu.VMEM((b_w, _BLK), jnp.float32),
            pltpu.VMEM((_HB, d), jnp.float32),
            pltpu.VMEM((_HB, d), jnp.float32),
            pltpu.VMEM_SHARED((n_pad, d), jnp.float32),
            pltpu.SemaphoreType.DMA,
            pltpu.SemaphoreType.DMA,
            pltpu.SemaphoreType.DMA,
            pltpu.SemaphoreType.DMA,
        ],
    )
    def body(x_hbm, col_hbm, row_hbm, val_hbm, z_hbm, out_hbm,
             col_v, row_v, val_v, rows_a, rows_b, acc_sh,
             gsem_a, gsem_b, ssem_a, ssem_b):
        cid = lax.axis_index("c")
        sid = lax.axis_index("s")
        wid = sid * _NC + cid
        sub_rows = pl.ds(sid * rows_per_sub, rows_per_sub)

        pltpu.sync_copy(z_hbm, acc_sh.at[sub_rows])
        pltpu.sync_copy(col_hbm.at[wid], col_v)
        pltpu.sync_copy(row_hbm.at[wid], row_v)
        pltpu.sync_copy(val_hbm.at[wid], val_v)
        plsc.subcore_barrier()

        def scale(j, h, rows):
            def scale16(g, c2):
                vv = val_v[j, pl.ds(h * _HB + g * _LANES, _LANES)]
                base = g * _LANES
                for kk in range(_LANES):
                    v = vv[kk]
                    for c in range(d // _LANES):
                        sl = pl.ds(c * _LANES, _LANES)
                        rows[base + kk, sl] = rows[base + kk, sl] * v
                return c2

            lax.fori_loop(0, _HB // _LANES, scale16, 0)

        def gath(j, h, rows_ref, sem):
            return pltpu.async_copy(
                x_hbm.at[col_v.at[j, pl.ds(h * _HB, _HB)]], rows_ref, sem)

        def scat(j, h, rows_ref, sem):
            return pltpu.async_copy(
                rows_ref, acc_sh.at[row_v.at[j, pl.ds(h * _HB, _HB)]], sem,
                add=True)

        def process_rowpair(jj, carry):
            j0 = jj * 2
            j1 = j0 + 1
            ga = gath(j0, 0, rows_a, gsem_a)
            gb = gath(j0, 1, rows_b, gsem_b)
            ga.wait()
            scale(j0, 0, rows_a)
            sa = scat(j0, 0, rows_a, ssem_a)
            gb.wait()
            scale(j0, 1, rows_b)
            sb = scat(j0, 1, rows_b, ssem_b)
            sa.wait()
            ga = gath(j1, 0, rows_a, gsem_a)
            sb.wait()
            gb = gath(j1, 1, rows_b, gsem_b)
            ga.wait()
            scale(j1, 0, rows_a)
            sa = scat(j1, 0, rows_a, ssem_a)
            gb.wait()
            scale(j1, 1, rows_b)
            sb = scat(j1, 1, rows_b, ssem_b)
            sa.wait()
            sb.wait()
            return carry

        lax.fori_loop(0, b_w // 2, process_rowpair, 0)

        plsc.subcore_barrier()
        pltpu.sync_copy(acc_sh.at[sub_rows], out_hbm.at[cid].at[sub_rows])

    return body(x, col_p, row_p, val_p, zeros)


def _mm_body(p0_ref, p1_ref, w_ref, b_ref, o_ref):
    s = p0_ref[...] + p1_ref[...]
    acc = jnp.dot(s, w_ref[...], preferred_element_type=jnp.float32)
    o_ref[...] = jnp.maximum(acc + b_ref[...], 0.0)


def _tc_finish(partial, W, b, n):
    d_in = partial.shape[2]
    d_out = W.shape[1]
    bm = 2000 if n % 2000 == 0 else n
    return pl.pallas_call(
        _mm_body,
        grid=(n // bm,),
        in_specs=[
            pl.BlockSpec((bm, d_in), lambda i: (i, 0)),
            pl.BlockSpec((bm, d_in), lambda i: (i, 0)),
            pl.BlockSpec((d_in, d_out), lambda i: (0, 0)),
            pl.BlockSpec((1, d_out), lambda i: (0, 0)),
        ],
        out_specs=pl.BlockSpec((bm, d_out), lambda i: (i, 0)),
        out_shape=jax.ShapeDtypeStruct((n, d_out), jnp.float32),
    )(partial[0], partial[1], W, b.reshape(1, d_out))


def kernel(x, adj_indices, adj_values, W, b):
    n, d = x.shape
    e = adj_values.shape[0]
    n_blocks = pl.cdiv(e, _BLK)
    b_w = pl.cdiv(n_blocks, _NW)
    b_w = b_w + (b_w % 2)
    pad = _NW * b_w * _BLK - e
    col_p = jnp.pad(adj_indices[1], (0, pad)).reshape(_NW, b_w, _BLK)
    row_p = jnp.pad(adj_indices[0], (0, pad)).reshape(_NW, b_w, _BLK)
    val_p = jnp.pad(adj_values, (0, pad)).reshape(_NW, b_w, _BLK)
    n_pad = ((n + 8 * _NS - 1) // (8 * _NS)) * (8 * _NS)
    zeros = jnp.zeros((n_pad // _NS, d), jnp.float32)
    partial = _sc_aggregate(x, col_p, row_p, val_p, zeros, b_w, n_pad)
    return _tc_finish(partial, W, b, n)


# speedup vs baseline: 1.4692x; 1.4692x over previous
import functools

import jax
import jax.numpy as jnp
from jax import lax
from jax.experimental import pallas as pl
from jax.experimental.pallas import tpu as pltpu
from jax.experimental.pallas import tpu_sc as plsc

_NC = 2
_NS = 16
_NW = _NC * _NS
_LANES = 16
_BLK = 128
_HB = 64


def _sc_aggregate(x, col_p, row_p, val_p, zeros, b_w, n_pad):
    _, d = x.shape
    rows_per_sub = n_pad // _NS
    mesh = plsc.VectorSubcoreMesh(core_axis_name="c", subcore_axis_name="s")

    @functools.partial(
        pl.kernel,
        out_type=jax.ShapeDtypeStruct((_NC, n_pad, d), jnp.float32),
        mesh=mesh,
        scratch_types=[
            pltpu.VMEM((b_w, _BLK), jnp.int32),
            pltpu.VMEM((b_w, _BLK), jnp.int32),
            pltpu.VMEM((b_w, _BLK), jnp.float32),
            pltpu.VMEM((_HB, d), jnp.float32),
            pltpu.VMEM((_HB, d), jnp.float32),
            pltpu.VMEM_SHARED((n_pad, d), jnp.float32),
            pltpu.SemaphoreType.DMA,
            pltpu.SemaphoreType.DMA,
            pltpu.SemaphoreType.DMA,
            pltpu.SemaphoreType.DMA,
        ],
    )
    def body(x_hbm, col_hbm, row_hbm, val_hbm, z_hbm, out_hbm,
             col_v, row_v, val_v, rows_a, rows_b, acc_sh,
             gsem_a, gsem_b, ssem_a, ssem_b):
        cid = lax.axis_index("c")
        sid = lax.axis_index("s")
        wid = sid * _NC + cid
        sub_rows = pl.ds(sid * rows_per_sub, rows_per_sub)

        pltpu.sync_copy(z_hbm, acc_sh.at[sub_rows])
        pltpu.sync_copy(col_hbm.at[wid], col_v)
        pltpu.sync_copy(row_hbm.at[wid], row_v)
        pltpu.sync_copy(val_hbm.at[wid], val_v)
        plsc.subcore_barrier()

        def scale(j, h, rows):
            def scale16(g, c2):
                vv = val_v[j, pl.ds(h * _HB + g * _LANES, _LANES)]
                base = g * _LANES
                for kk in range(_LANES):
                    v = vv[kk]
                    for c in range(d // _LANES):
                        sl = pl.ds(c * _LANES, _LANES)
                        rows[base + kk, sl] = rows[base + kk, sl] * v
                return c2

            lax.fori_loop(0, _HB // _LANES, scale16, 0)

        def gath(j, h, rows_ref, sem):
            return pltpu.async_copy(
                x_hbm.at[col_v.at[j, pl.ds(h * _HB, _HB)]], rows_ref, sem)

        def scat(j, h, rows_ref, sem):
            return pltpu.async_copy(
                rows_ref, acc_sh.at[row_v.at[j, pl.ds(h * _HB, _HB)]], sem,
                add=True)

        def process_block(j, carry):
            ga = gath(j, 0, rows_a, gsem_a)
            gb = gath(j, 1, rows_b, gsem_b)
            ga.wait()
            scale(j, 0, rows_a)
            sa = scat(j, 0, rows_a, ssem_a)
            gb.wait()
            scale(j, 1, rows_b)
            sb = scat(j, 1, rows_b, ssem_b)
            sa.wait()
            sb.wait()
            return carry

        lax.fori_loop(0, b_w, process_block, 0)

        plsc.subcore_barrier()
        pltpu.sync_copy(acc_sh.at[sub_rows], out_hbm.at[cid].at[sub_rows])

    return body(x, col_p, row_p, val_p, zeros)


def _mm_body(p0_ref, p1_ref, w_ref, b_ref, o_ref):
    s = p0_ref[...] + p1_ref[...]
    acc = jnp.dot(s, w_ref[...], preferred_element_type=jnp.float32)
    o_ref[...] = jnp.maximum(acc + b_ref[...], 0.0)


def _tc_finish(partial, W, b, n):
    d_in = partial.shape[2]
    d_out = W.shape[1]
    bm = 1000 if n % 1000 == 0 else n
    return pl.pallas_call(
        _mm_body,
        grid=(n // bm,),
        in_specs=[
            pl.BlockSpec((bm, d_in), lambda i: (i, 0)),
            pl.BlockSpec((bm, d_in), lambda i: (i, 0)),
            pl.BlockSpec((d_in, d_out), lambda i: (0, 0)),
            pl.BlockSpec((1, d_out), lambda i: (0, 0)),
        ],
        out_specs=pl.BlockSpec((bm, d_out), lambda i: (i, 0)),
        out_shape=jax.ShapeDtypeStruct((n, d_out), jnp.float32),
    )(partial[0], partial[1], W, b.reshape(1, d_out))


def kernel(x, adj_indices, adj_values, W, b):
    n, d = x.shape
    e = adj_values.shape[0]
    n_blocks = pl.cdiv(e, _BLK)
    b_w = pl.cdiv(n_blocks, _NW)
    pad = _NW * b_w * _BLK - e
    col_p = jnp.pad(adj_indices[1], (0, pad)).reshape(_NW, b_w, _BLK)
    row_p = jnp.pad(adj_indices[0], (0, pad)).reshape(_NW, b_w, _BLK)
    val_p = jnp.pad(adj_values, (0, pad)).reshape(_NW, b_w, _BLK)
    n_pad = ((n + 8 * _NS - 1) // (8 * _NS)) * (8 * _NS)
    zeros = jnp.zeros((n_pad // _NS, d), jnp.float32)
    partial = _sc_aggregate(x, col_p, row_p, val_p, zeros, b_w, n_pad)
    return _tc_finish(partial, W, b, n)


# speedup vs baseline: 1.4957x; 1.0180x over previous
import functools

import jax
import jax.numpy as jnp
from jax import lax
from jax.experimental import pallas as pl
from jax.experimental.pallas import tpu as pltpu
from jax.experimental.pallas import tpu_sc as plsc

_NC = 2
_NS = 16
_NW = _NC * _NS
_LANES = 16
_BLK = 128
_HB = 64


def _sc_aggregate(x, col_p, row_p, val_p, b_w, n_pad):
    _, d = x.shape
    rows_per_sub = n_pad // _NS
    mesh = plsc.VectorSubcoreMesh(core_axis_name="c", subcore_axis_name="s")

    @functools.partial(
        pl.kernel,
        out_type=jax.ShapeDtypeStruct((_NC, n_pad, d), jnp.float32),
        mesh=mesh,
        scratch_types=[
            pltpu.VMEM((b_w, _BLK), jnp.int32),
            pltpu.VMEM((b_w, _BLK), jnp.int32),
            pltpu.VMEM((b_w, _BLK), jnp.float32),
            pltpu.VMEM((_HB, d), jnp.float32),
            pltpu.VMEM((_HB, d), jnp.float32),
            pltpu.VMEM_SHARED((n_pad, d), jnp.float32),
            pltpu.SemaphoreType.DMA,
            pltpu.SemaphoreType.DMA,
            pltpu.SemaphoreType.DMA,
            pltpu.SemaphoreType.DMA,
        ],
    )
    def body(x_hbm, col_hbm, row_hbm, val_hbm, out_hbm,
             col_v, row_v, val_v, rows_a, rows_b, acc_sh,
             gsem_a, gsem_b, ssem_a, ssem_b):
        cid = lax.axis_index("c")
        sid = lax.axis_index("s")
        wid = sid * _NC + cid
        sub_rows = pl.ds(sid * rows_per_sub, rows_per_sub)

        def zrow(r, c2):
            for c in range(d // _LANES):
                rows_a[r, pl.ds(c * _LANES, _LANES)] = jnp.zeros(
                    (_LANES,), jnp.float32)
            return c2

        lax.fori_loop(0, _HB, zrow, 0)
        n_zc = rows_per_sub // _HB
        rem = rows_per_sub - n_zc * _HB

        def zcopy(i, c2):
            pltpu.sync_copy(
                rows_a, acc_sh.at[pl.ds(sid * rows_per_sub + i * _HB, _HB)])
            return c2

        lax.fori_loop(0, n_zc, zcopy, 0)
        if rem:
            pltpu.sync_copy(
                rows_a.at[pl.ds(0, rem)],
                acc_sh.at[pl.ds(sid * rows_per_sub + n_zc * _HB, rem)])
        pltpu.sync_copy(col_hbm.at[wid], col_v)
        pltpu.sync_copy(row_hbm.at[wid], row_v)
        pltpu.sync_copy(val_hbm.at[wid], val_v)
        plsc.subcore_barrier()

        def scale(j, h, rows):
            def scale16(g, c2):
                vv = val_v[j, pl.ds(h * _HB + g * _LANES, _LANES)]
                base = g * _LANES
                for kk in range(_LANES):
                    v = vv[kk]
                    for c in range(d // _LANES):
                        sl = pl.ds(c * _LANES, _LANES)
                        rows[base + kk, sl] = rows[base + kk, sl] * v
                return c2

            lax.fori_loop(0, _HB // _LANES, scale16, 0)

        def gath(j, h, rows_ref, sem):
            return pltpu.async_copy(
                x_hbm.at[col_v.at[j, pl.ds(h * _HB, _HB)]], rows_ref, sem)

        def scat(j, h, rows_ref, sem):
            return pltpu.async_copy(
                rows_ref, acc_sh.at[row_v.at[j, pl.ds(h * _HB, _HB)]], sem,
                add=True)

        def process_block(j, carry):
            ga = gath(j, 0, rows_a, gsem_a)
            gb = gath(j, 1, rows_b, gsem_b)
            ga.wait()
            scale(j, 0, rows_a)
            sa = scat(j, 0, rows_a, ssem_a)
            gb.wait()
            scale(j, 1, rows_b)
            sb = scat(j, 1, rows_b, ssem_b)
            sa.wait()
            sb.wait()
            return carry

        lax.fori_loop(0, b_w, process_block, 0)

        plsc.subcore_barrier()
        pltpu.sync_copy(acc_sh.at[sub_rows], out_hbm.at[cid].at[sub_rows])

    return body(x, col_p, row_p, val_p)


def _mm_body(p0_ref, p1_ref, w_ref, b_ref, o_ref):
    s = p0_ref[...] + p1_ref[...]
    acc = jnp.dot(s, w_ref[...], preferred_element_type=jnp.float32)
    o_ref[...] = jnp.maximum(acc + b_ref[...], 0.0)


def _tc_finish(partial, W, b, n):
    d_in = partial.shape[2]
    d_out = W.shape[1]
    bm = 2000 if n % 2000 == 0 else (1000 if n % 1000 == 0 else n)
    return pl.pallas_call(
        _mm_body,
        grid=(n // bm,),
        in_specs=[
            pl.BlockSpec((bm, d_in), lambda i: (i, 0)),
            pl.BlockSpec((bm, d_in), lambda i: (i, 0)),
            pl.BlockSpec((d_in, d_out), lambda i: (0, 0)),
            pl.BlockSpec((1, d_out), lambda i: (0, 0)),
        ],
        out_specs=pl.BlockSpec((bm, d_out), lambda i: (i, 0)),
        out_shape=jax.ShapeDtypeStruct((n, d_out), jnp.float32),
    )(partial[0], partial[1], W, b.reshape(1, d_out))


def kernel(x, adj_indices, adj_values, W, b):
    n, d = x.shape
    e = adj_values.shape[0]
    n_blocks = pl.cdiv(e, _BLK)
    b_w = pl.cdiv(n_blocks, _NW)
    pad = _NW * b_w * _BLK - e
    col_p = jnp.pad(adj_indices[1], (0, pad)).reshape(_NW, b_w, _BLK)
    row_p = jnp.pad(adj_indices[0], (0, pad)).reshape(_NW, b_w, _BLK)
    val_p = jnp.pad(adj_values, (0, pad)).reshape(_NW, b_w, _BLK)
    n_pad = ((n + 8 * _NS - 1) // (8 * _NS)) * (8 * _NS)
    partial = _sc_aggregate(x, col_p, row_p, val_p, b_w, n_pad)
    return _tc_finish(partial, W, b, n)
